# Initial kernel scaffold; baseline (speedup 1.0000x reference)
import jax
import jax.numpy as jnp
from jax.experimental import pallas as pl


def kernel(node_repr, H_case, Wc, bc, W1, b1, W2, b2, ln_g, ln_b):
    raise NotImplementedError("write your pallas kernel here")


# speedup vs baseline: 6.3082x; 6.3082x over previous
import jax
import jax.numpy as jnp
from jax import lax
from jax.experimental import pallas as pl


def _ctx_kernel(ht_ref, x1_ref, w1b_ref, wc_ref, bc_ref, b1_ref,
                ctx_ref, u1_ref, cu_ref):
    d = w1b_ref.shape[0]
    S = jnp.dot(ht_ref[...], x1_ref[...], preferred_element_type=jnp.float32)
    ctx = S[:, :d] / jnp.maximum(S[:, d:d + 1], 1e-8)
    ctx_ref[...] = ctx
    u1_ref[...] = jnp.dot(ctx, w1b_ref[...],
                          preferred_element_type=jnp.float32) + b1_ref[...]
    cu_ref[...] = jnp.dot(ctx, wc_ref[...],
                          preferred_element_type=jnp.float32) + bc_ref[...]


def _edge_kernel(x_ref, ctx_ref, u1_ref, cu_ref, w1a_ref, w1cd_ref,
                 w2_ref, b2_ref, lng_ref, lnb_ref, out_ref):
    X = x_ref[...]
    P = jnp.dot(X, w1a_ref[...], preferred_element_type=jnp.float32)
    num_case = ctx_ref.shape[0]

    def body(j, _):
        c = ctx_ref[pl.ds(j, 1), :]
        u1 = u1_ref[pl.ds(j, 1), :]
        cu = cu_ref[pl.ds(j, 1), :]
        Y = jnp.concatenate([X * c, jnp.abs(X - c)], axis=1)
        G = P + u1 + jnp.dot(Y, w1cd_ref[...],
                             preferred_element_type=jnp.float32)
        h = jnp.maximum(G, 0.0)
        logits = jnp.dot(h, w2_ref[...],
                         preferred_element_type=jnp.float32) + b2_ref[...]
        t = 0.3 * jax.nn.sigmoid(logits)
        pre = X + t * (cu - X)
        mu = jnp.mean(pre, axis=1, keepdims=True)
        xc = pre - mu
        var = jnp.mean(xc * xc, axis=1, keepdims=True)
        out = xc * lax.rsqrt(var + 1e-5) * lng_ref[...] + lnb_ref[...]
        out_ref[:, pl.ds(j, 1), :] = out[:, None, :]
        return 0

    lax.fori_loop(0, num_case, body, 0)


def kernel(node_repr, H_case, Wc, bc, W1, b1, W2, b2, ln_g, ln_b):
    num_hpo, d = node_repr.shape
    num_case = H_case.shape[1]
    dm = W1.shape[1]

    X1 = jnp.concatenate(
        [node_repr, jnp.ones((num_hpo, 1), dtype=node_repr.dtype)], axis=1)
    Ht = H_case.T

    ctx, u1, cu = pl.pallas_call(
        _ctx_kernel,
        out_shape=[
            jax.ShapeDtypeStruct((num_case, d), jnp.float32),
            jax.ShapeDtypeStruct((num_case, dm), jnp.float32),
            jax.ShapeDtypeStruct((num_case, d), jnp.float32),
        ],
    )(Ht, X1, W1[d:2 * d], Wc, bc.reshape(1, d), b1.reshape(1, dm))

    NB = 400
    grid = (num_hpo // NB,)
    out = pl.pallas_call(
        _edge_kernel,
        grid=grid,
        in_specs=[
            pl.BlockSpec((NB, d), lambda b: (b, 0)),
            pl.BlockSpec((num_case, d), lambda b: (0, 0)),
            pl.BlockSpec((num_case, dm), lambda b: (0, 0)),
            pl.BlockSpec((num_case, d), lambda b: (0, 0)),
            pl.BlockSpec((d, dm), lambda b: (0, 0)),
            pl.BlockSpec((2 * d, dm), lambda b: (0, 0)),
            pl.BlockSpec((dm, d), lambda b: (0, 0)),
            pl.BlockSpec((1, d), lambda b: (0, 0)),
            pl.BlockSpec((1, d), lambda b: (0, 0)),
            pl.BlockSpec((1, d), lambda b: (0, 0)),
        ],
        out_specs=pl.BlockSpec((NB, num_case, d), lambda b: (b, 0, 0)),
        out_shape=jax.ShapeDtypeStruct((num_hpo, num_case, d), jnp.float32),
    )(node_repr, ctx, u1, cu, W1[:d], W1[2 * d:], W2,
      b2.reshape(1, d), ln_g.reshape(1, d), ln_b.reshape(1, d))

    return out.reshape(num_hpo * num_case, d)


# speedup vs baseline: 8.2199x; 1.3030x over previous
import jax
import jax.numpy as jnp
from jax import lax
from jax.experimental import pallas as pl


def _ctx_kernel(ht_ref, x1_ref, w1b_ref, wc_ref, bc_ref, b1_ref,
                ctx_ref, u1_ref, cu_ref):
    d = w1b_ref.shape[0]
    num_case = ht_ref.shape[0]
    S = jnp.dot(ht_ref[...], x1_ref[...], preferred_element_type=jnp.float32)
    ctx = S[:, :d] / jnp.maximum(S[:, d:d + 1], 1e-8)
    u1 = jnp.dot(ctx, w1b_ref[...],
                 preferred_element_type=jnp.float32) + b1_ref[...]
    cu = jnp.dot(ctx, wc_ref[...],
                 preferred_element_type=jnp.float32) + bc_ref[...]
    rep = (num_case, 8, d)
    ctx_ref[...] = jnp.broadcast_to(ctx[:, None, :], rep)
    u1_ref[...] = jnp.broadcast_to(u1[:, None, :], rep)
    cu_ref[...] = jnp.broadcast_to(cu[:, None, :], rep)


def _edge_kernel(x_ref, ctx_ref, u1_ref, cu_ref, w1a_ref, w1c_ref, w1d_ref,
                 w2_ref, b2_ref, lng_ref, lnb_ref, out_ref):
    X = x_ref[...]
    nb, d = X.shape
    X3 = X.reshape(nb // 8, 8, d)
    P = jnp.dot(X, w1a_ref[...], preferred_element_type=jnp.float32)
    num_case = ctx_ref.shape[0]
    ones_d = jnp.full((d, d), 1.0 / d, dtype=jnp.float32)

    for j in range(num_case):
        c = ctx_ref[j, :, :][None]
        u1 = u1_ref[j, :, :][None]
        cu = cu_ref[j, :, :][None]
        Y1 = (X3 * c).reshape(nb, d)
        Y2 = jnp.abs(X3 - c).reshape(nb, d)
        G = (P
             + jnp.dot(Y1, w1c_ref[...], preferred_element_type=jnp.float32)
             + jnp.dot(Y2, w1d_ref[...], preferred_element_type=jnp.float32)
             ).reshape(nb // 8, 8, d) + u1
        h = jnp.maximum(G.reshape(nb, d), 0.0)
        logits = jnp.dot(h, w2_ref[...],
                         preferred_element_type=jnp.float32) + b2_ref[...]
        t = 0.3 * jax.nn.sigmoid(logits)
        pre = X + (t.reshape(nb // 8, 8, d) * (cu - X3)).reshape(nb, d)
        mu = jnp.dot(pre, ones_d, preferred_element_type=jnp.float32)
        msq = jnp.dot(pre * pre, ones_d, preferred_element_type=jnp.float32)
        var = msq - mu * mu
        out = (pre - mu) * lax.rsqrt(var + 1e-5) * lng_ref[...] + lnb_ref[...]
        out_ref[:, j, :] = out


def kernel(node_repr, H_case, Wc, bc, W1, b1, W2, b2, ln_g, ln_b):
    num_hpo, d = node_repr.shape
    num_case = H_case.shape[1]
    dm = W1.shape[1]

    X1 = jnp.concatenate(
        [node_repr, jnp.ones((num_hpo, 1), dtype=node_repr.dtype)], axis=1)
    Ht = H_case.T

    ctx, u1, cu = pl.pallas_call(
        _ctx_kernel,
        out_shape=[
            jax.ShapeDtypeStruct((num_case, 8, d), jnp.float32),
            jax.ShapeDtypeStruct((num_case, 8, dm), jnp.float32),
            jax.ShapeDtypeStruct((num_case, 8, d), jnp.float32),
        ],
    )(Ht, X1, W1[d:2 * d], Wc, bc.reshape(1, d), b1.reshape(1, dm))

    NB = 400
    grid = (num_hpo // NB,)
    out = pl.pallas_call(
        _edge_kernel,
        grid=grid,
        in_specs=[
            pl.BlockSpec((NB, d), lambda b: (b, 0)),
            pl.BlockSpec((num_case, 8, d), lambda b: (0, 0, 0)),
            pl.BlockSpec((num_case, 8, dm), lambda b: (0, 0, 0)),
            pl.BlockSpec((num_case, 8, d), lambda b: (0, 0, 0)),
            pl.BlockSpec((d, dm), lambda b: (0, 0)),
            pl.BlockSpec((d, dm), lambda b: (0, 0)),
            pl.BlockSpec((d, dm), lambda b: (0, 0)),
            pl.BlockSpec((dm, d), lambda b: (0, 0)),
            pl.BlockSpec((1, d), lambda b: (0, 0)),
            pl.BlockSpec((1, d), lambda b: (0, 0)),
            pl.BlockSpec((1, d), lambda b: (0, 0)),
        ],
        out_specs=pl.BlockSpec((NB, num_case, d), lambda b: (b, 0, 0)),
        out_shape=jax.ShapeDtypeStruct((num_hpo, num_case, d), jnp.float32),
    )(node_repr, ctx, u1, cu, W1[:d], W1[2 * d:3 * d], W1[3 * d:], W2,
      b2.reshape(1, d), ln_g.reshape(1, d), ln_b.reshape(1, d))

    return out.reshape(num_hpo * num_case, d)
